# baseline (speedup 1.0000x reference)
import functools

import jax
import jax.numpy as jnp
from jax import lax
from jax.experimental import pallas as pl
from jax.experimental.pallas import tpu as pltpu
from jax.experimental.pallas import tpu_sc as plsc

NC = 2
NS = 16
NW = NC * NS
B = 16384
S = 50
D = 64
JTILE = 128
BLK_PER_W = (B // JTILE) // NW
STEPS = BLK_PER_W * S
SLAB = B // NW * S


def _gather_kernel():
    mesh = plsc.VectorSubcoreMesh(core_axis_name="c", subcore_axis_name="s")

    @functools.partial(
        pl.kernel,
        mesh=mesh,
        out_type=jax.ShapeDtypeStruct((S, D // 8, B // JTILE, 8 * JTILE),
                                      jnp.float32),
        scratch_types=(
            [pltpu.VMEM((SLAB,), jnp.int32)]
            + [pltpu.VMEM((JTILE,), jnp.int32) for _ in range(2)]
            + [pltpu.VMEM((JTILE, D), jnp.float32) for _ in range(2)]
            + [pltpu.VMEM((D * JTILE,), jnp.float32) for _ in range(2)]
            + [pltpu.SemaphoreType.DMA for _ in range(4)]
        ),
        compiler_params=pltpu.CompilerParams(
            use_tc_tiling_on_sc=False, needs_layout_passes=False,
            disable_bounds_checks=True),
    )
    def k(idx_hbm, table_hbm, out_hbm, idx_slab,
          ic0, ic1, rw0, rw1, tr0, tr1, gs0, gs1, ss0, ss1):
        idx_col = (ic0, ic1)
        rows = (rw0, rw1)
        tr = (tr0, tr1)
        gsem = (gs0, gs1)
        ssem = (ss0, ss1)
        wid = lax.axis_index("s") * NC + lax.axis_index("c")

        iota = lax.iota(jnp.int32, 16)
        iota_s = iota * S
        cvec_tr = [(iota + cg * 16) * JTILE for cg in range(D // 16)]

        pltpu.sync_copy(idx_hbm.at[pl.ds(wid * SLAB, SLAB)], idx_slab)

        def build_idx(buf, d0b, s):
            base = d0b * (JTILE * S) + s
            for jg in range(JTILE // 16):
                vec = iota_s + (base + jg * 16 * S)
                buf[pl.ds(jg * 16, 16)] = plsc.load_gather(idx_slab, [vec])

        def fire_gather(p):
            pltpu.async_copy(table_hbm.at[idx_col[p]], rows[p], gsem[p])

        def wait_gather(p):
            pltpu.make_async_copy(
                table_hbm.at[idx_col[p]], rows[p], gsem[p]).wait()

        def transpose(p):
            rows_p, tr_p = rows[p], tr[p]

            @plsc.parallel_loop(0, JTILE, 1, unroll=8)
            def _(j):
                for cg in range(D // 16):
                    vals = rows_p[j, pl.ds(cg * 16, 16)]
                    plsc.store_scatter(tr_p, [cvec_tr[cg] + j], vals)

        def fire_stores(p, s, gd0b):
            for d2b in range(D // 8):
                pltpu.async_copy(
                    tr[p].at[pl.ds(d2b * 8 * JTILE, 8 * JTILE)],
                    out_hbm.at[s, d2b, gd0b], ssem[p])

        def drain_stores(p):
            for d2b in range(D // 8):
                pltpu.make_async_copy(
                    out_hbm.at[0, d2b, 0],
                    tr[p].at[pl.ds(d2b * 8 * JTILE, 8 * JTILE)],
                    ssem[p]).wait()

        def advance(s, d0b):
            s1 = s + 1
            wrap = s1 == S
            return jnp.where(wrap, 0, s1), jnp.where(wrap, d0b + 1, d0b)

        build_idx(idx_col[0], 0, 0)
        fire_gather(0)

        def body(g, carry):
            s, d0b = carry
            for p in range(2):
                t = g * 2 + p
                s_n, d0b_n = advance(s, d0b)

                @pl.when(t < STEPS - 1)
                def _():
                    build_idx(idx_col[1 - p], d0b_n, s_n)
                    fire_gather(1 - p)

                wait_gather(p)

                @pl.when(t >= 2)
                def _():
                    drain_stores(p)

                transpose(p)
                fire_stores(p, s, wid * BLK_PER_W + d0b)
                s, d0b = s_n, d0b_n
            return s, d0b

        lax.fori_loop(0, STEPS // 2, body,
                      (jnp.int32(0), jnp.int32(0)))

        drain_stores(0)
        drain_stores(1)

    return k


def kernel(token_ids, embedding_matrix):
    idx = token_ids.reshape(-1).astype(jnp.int32)
    out4 = _gather_kernel()(idx, embedding_matrix)
    out5 = out4.reshape(S, D // 8, B // JTILE, 8, JTILE)
    return out5.transpose((2, 4, 0, 1, 3)).reshape(B, S, D)


# speedup vs baseline: 1.6671x; 1.6671x over previous
import functools

import jax
import jax.numpy as jnp
from jax import lax
from jax.experimental import pallas as pl
from jax.experimental.pallas import tpu as pltpu
from jax.experimental.pallas import tpu_sc as plsc

NC = 2
NS = 16
NW = NC * NS
B = 16384
S = 50
D = 64
JTILE = 128
BLK_PER_W = (B // JTILE) // NW
STEPS = BLK_PER_W * S
SLAB = B // NW * S
TRW = JTILE + 1


def _gather_kernel():
    mesh = plsc.VectorSubcoreMesh(core_axis_name="c", subcore_axis_name="s")

    @functools.partial(
        pl.kernel,
        mesh=mesh,
        out_type=jax.ShapeDtypeStruct((S, D // 8, B // JTILE, 8, JTILE),
                                      jnp.float32),
        scratch_types=(
            [pltpu.VMEM((SLAB,), jnp.int32)]
            + [pltpu.VMEM((JTILE,), jnp.int32) for _ in range(2)]
            + [pltpu.VMEM((JTILE, D), jnp.float32) for _ in range(2)]
            + [pltpu.VMEM((D, TRW), jnp.float32) for _ in range(2)]
            + [pltpu.SemaphoreType.DMA for _ in range(4)]
        ),
        compiler_params=pltpu.CompilerParams(
            use_tc_tiling_on_sc=False, needs_layout_passes=False,
            disable_bounds_checks=True),
    )
    def k(idx_hbm, table_hbm, out_hbm, idx_slab,
          ic0, ic1, rw0, rw1, tr0, tr1, gs0, gs1, ss0, ss1):
        idx_col = (ic0, ic1)
        rows = (rw0, rw1)
        tr = (tr0, tr1)
        gsem = (gs0, gs1)
        ssem = (ss0, ss1)
        wid = lax.axis_index("s") * NC + lax.axis_index("c")

        iota = lax.iota(jnp.int32, 16)
        iota_s = iota * S
        cvec_tr = [iota + cg * 16 for cg in range(D // 16)]

        pltpu.sync_copy(idx_hbm.at[pl.ds(wid * SLAB, SLAB)], idx_slab)

        def build_idx(buf, d0b, s):
            base = d0b * (JTILE * S) + s
            for jg in range(JTILE // 16):
                vec = iota_s + (base + jg * 16 * S)
                buf[pl.ds(jg * 16, 16)] = plsc.load_gather(idx_slab, [vec])

        def fire_gather(p):
            pltpu.async_copy(table_hbm.at[idx_col[p]], rows[p], gsem[p])

        def wait_gather(p):
            pltpu.make_async_copy(
                table_hbm.at[idx_col[p]], rows[p], gsem[p]).wait()

        def transpose(p):
            rows_p, tr_p = rows[p], tr[p]

            @plsc.parallel_loop(0, JTILE, 1, unroll=8)
            def _(j):
                jv = jnp.zeros((16,), jnp.int32) + j
                for cg in range(D // 16):
                    vals = rows_p[j, pl.ds(cg * 16, 16)]
                    plsc.store_scatter(tr_p, [cvec_tr[cg], jv], vals)

        def fire_stores(p, s, gd0b):
            for d2b in range(D // 8):
                pltpu.async_copy(
                    tr[p].at[pl.ds(d2b * 8, 8), pl.ds(0, JTILE)],
                    out_hbm.at[s, d2b, gd0b], ssem[p])

        def drain_stores(p):
            for d2b in range(D // 8):
                pltpu.make_async_copy(
                    out_hbm.at[0, d2b, 0],
                    tr[p].at[pl.ds(d2b * 8, 8), pl.ds(0, JTILE)],
                    ssem[p]).wait()

        def advance(s, d0b):
            s1 = s + 1
            wrap = s1 == S
            return jnp.where(wrap, 0, s1), jnp.where(wrap, d0b + 1, d0b)

        build_idx(idx_col[0], 0, 0)
        fire_gather(0)

        def body(g, carry):
            s, d0b = carry
            for p in range(2):
                t = g * 2 + p
                s_n, d0b_n = advance(s, d0b)

                @pl.when(t < STEPS - 1)
                def _():
                    build_idx(idx_col[1 - p], d0b_n, s_n)
                    fire_gather(1 - p)

                wait_gather(p)

                @pl.when(t >= 2)
                def _():
                    drain_stores(p)

                transpose(p)
                fire_stores(p, s, wid * BLK_PER_W + d0b)
                s, d0b = s_n, d0b_n
            return s, d0b

        lax.fori_loop(0, STEPS // 2, body,
                      (jnp.int32(0), jnp.int32(0)))

        drain_stores(0)
        drain_stores(1)

    return k


def kernel(token_ids, embedding_matrix):
    idx = token_ids.reshape(-1).astype(jnp.int32)
    out5 = _gather_kernel()(idx, embedding_matrix)
    return out5.transpose((2, 4, 0, 1, 3)).reshape(B, S, D)
